# baseline (speedup 1.0000x reference)
import functools

import jax
import jax.numpy as jnp
from jax import lax
from jax.experimental import pallas as pl
from jax.experimental.pallas import tpu as pltpu
from jax.experimental.pallas import tpu_sc as plsc

B, C, H, W = 4, 32, 512, 512
HW = H * W
N = 100000
NQ = B * N
NWK = 32
SB = 128
NBLK = NQ // SB
ROUNDS = -(-NBLK // NWK)


def _sc_body(table, qxy, out, qb_v, idx_v, w_v,
             rows0, rows1, rows2, rows3, out_v, sem):
    rows = (rows0, rows1, rows2, rows3)
    cid = lax.axis_index("c")
    sid = lax.axis_index("s")
    wid = sid * 2 + cid

    @pl.loop(0, ROUNDS)
    def do_round(r):
        blk = wid + r * NWK

        @pl.when(blk < NBLK)
        def do_block():
            lane = lax.iota(jnp.int32, 16)
            pltpu.sync_copy(qxy.at[blk], qb_v)
            for g in range(SB // 16):
                gx = qb_v[pl.ds(g * 16, 16)]
                gy = qb_v[pl.ds(SB + g * 16, 16)]
                ix = ((gx + 1.0) * W - 1.0) * 0.5
                iy = ((gy + 1.0) * H - 1.0) * 0.5
                ix0 = (ix + 1.0).astype(jnp.int32) - 1
                iy0 = (iy + 1.0).astype(jnp.int32) - 1
                wx1 = ix - ix0.astype(jnp.float32)
                wy1 = iy - iy0.astype(jnp.float32)
                wx0 = 1.0 - wx1
                wy0 = 1.0 - wy1
                ix1 = ix0 + 1
                iy1 = iy0 + 1
                zero = gx * 0.0
                wx0 = jnp.where(ix0 >= 0, wx0, zero)
                wx1 = jnp.where(ix1 <= W - 1, wx1, zero)
                wy0 = jnp.where(iy0 >= 0, wy0, zero)
                wy1 = jnp.where(iy1 <= H - 1, wy1, zero)
                cx0 = jnp.maximum(ix0, 0)
                cx1 = jnp.minimum(ix1, W - 1)
                cy0 = jnp.maximum(iy0, 0)
                cy1 = jnp.minimum(iy1, H - 1)
                gq = blk * SB + g * 16 + lane
                tb = (gq // N) * HW
                gsl = pl.ds(g * 16, 16)
                r0 = tb + cy0 * W
                r1 = tb + cy1 * W
                idx_v[0, gsl] = r0 + cx0
                idx_v[1, gsl] = r0 + cx1
                idx_v[2, gsl] = r1 + cx0
                idx_v[3, gsl] = r1 + cx1
                w_v[0, gsl] = wy0 * wx0
                w_v[1, gsl] = wy0 * wx1
                w_v[2, gsl] = wy1 * wx0
                w_v[3, gsl] = wy1 * wx1
            cps = [pltpu.async_copy(table.at[idx_v.at[c]], rows[c], sem)
                   for c in range(4)]
            for cp in cps:
                cp.wait()

            for g in range(SB // 16):
                w0 = w_v[0, pl.ds(g * 16, 16)]
                w1 = w_v[1, pl.ds(g * 16, 16)]
                w2 = w_v[2, pl.ds(g * 16, 16)]
                w3 = w_v[3, pl.ds(g * 16, 16)]
                for q in range(16):
                    qq = g * 16 + q
                    for h in range(C // 16):
                        hsl = pl.ds(h * 16, 16)
                        out_v[qq, hsl] = (rows0[qq, hsl] * w0[q]
                                          + rows1[qq, hsl] * w1[q]
                                          + rows2[qq, hsl] * w2[q]
                                          + rows3[qq, hsl] * w3[q])

            pltpu.sync_copy(out_v, out.at[pl.ds(blk * SB, SB)])


@jax.jit
def kernel(x, query_pos):
    table = (x.transpose(0, 2, 3, 1)
             .reshape(B, H, W // 4, 4 * C)
             .reshape(B * HW, C))
    gx = query_pos[..., 1].reshape(NBLK, SB)
    gy = query_pos[..., 0].reshape(NBLK, SB)
    qxy = jnp.concatenate([gx, gy], axis=1)

    mesh = plsc.VectorSubcoreMesh(core_axis_name="c", subcore_axis_name="s")
    run = functools.partial(
        pl.kernel,
        mesh=mesh,
        out_type=jax.ShapeDtypeStruct((NQ, C), jnp.float32),
        compiler_params=pltpu.CompilerParams(
            use_tc_tiling_on_sc=False, needs_layout_passes=False),
        scratch_types=[
            pltpu.VMEM((2 * SB,), jnp.float32),
            pltpu.VMEM((4, SB), jnp.int32),
            pltpu.VMEM((4, SB), jnp.float32),
            pltpu.VMEM((SB, C), jnp.float32),
            pltpu.VMEM((SB, C), jnp.float32),
            pltpu.VMEM((SB, C), jnp.float32),
            pltpu.VMEM((SB, C), jnp.float32),
            pltpu.VMEM((SB, C), jnp.float32),
            pltpu.SemaphoreType.DMA,
        ],
    )(_sc_body)
    return run(table, qxy)


# speedup vs baseline: 1.3833x; 1.3833x over previous
import functools

import jax
import jax.numpy as jnp
from jax import lax
from jax.experimental import pallas as pl
from jax.experimental.pallas import tpu as pltpu
from jax.experimental.pallas import tpu_sc as plsc

B, C, H, W = 4, 32, 512, 512
HW = H * W
N = 100000
NQ = B * N
NWK = 32
SB = 128
NBLK = NQ // SB
FULL = NBLK // NWK
EXTRA = NBLK - FULL * NWK
ROUNDS = FULL + 1


def _sc_body(table, qxy, out, qb, idx, wv, rows, out_v,
             gsem0, gsem1, osem0, osem1):
    gsem = (gsem0, gsem1)
    osem = (osem0, osem1)
    cid = lax.axis_index("c")
    sid = lax.axis_index("s")
    wid = sid * 2 + cid
    start = wid * FULL + jnp.minimum(wid, EXTRA)
    cnt = jnp.where(wid < EXTRA, FULL + 1, FULL)

    pltpu.sync_copy(qxy.at[pl.ds(start, FULL)], qb.at[pl.ds(0, FULL)])

    @pl.when(cnt == FULL + 1)
    def _():
        pltpu.sync_copy(qxy.at[pl.ds(start + FULL, 1)],
                        qb.at[pl.ds(FULL, 1)])

    def fire(t, s):
        @pl.when(t < cnt)
        def _():
            blk = start + t
            lane = lax.iota(jnp.int32, 16)
            for g in range(SB // 16):
                gx = qb[t, pl.ds(g * 16, 16)]
                gy = qb[t, pl.ds(SB + g * 16, 16)]
                ix = ((gx + 1.0) * W - 1.0) * 0.5
                iy = ((gy + 1.0) * H - 1.0) * 0.5
                ix0 = (ix + 1.0).astype(jnp.int32) - 1
                iy0 = (iy + 1.0).astype(jnp.int32) - 1
                wx1 = ix - ix0.astype(jnp.float32)
                wy1 = iy - iy0.astype(jnp.float32)
                wx0 = 1.0 - wx1
                wy0 = 1.0 - wy1
                ix1 = ix0 + 1
                iy1 = iy0 + 1
                zero = gx * 0.0
                wx0 = jnp.where(ix0 >= 0, wx0, zero)
                wx1 = jnp.where(ix1 <= W - 1, wx1, zero)
                wy0 = jnp.where(iy0 >= 0, wy0, zero)
                wy1 = jnp.where(iy1 <= H - 1, wy1, zero)
                cx0 = jnp.maximum(ix0, 0)
                cx1 = jnp.minimum(ix1, W - 1)
                cy0 = jnp.maximum(iy0, 0)
                cy1 = jnp.minimum(iy1, H - 1)
                gq = blk * SB + g * 16 + lane
                tb = (gq // N) * HW
                gsl = pl.ds(g * 16, 16)
                r0 = tb + cy0 * W
                r1 = tb + cy1 * W
                idx[s, 0, gsl] = r0 + cx0
                idx[s, 1, gsl] = r0 + cx1
                idx[s, 2, gsl] = r1 + cx0
                idx[s, 3, gsl] = r1 + cx1
                wv[s, 0, gsl] = wy0 * wx0
                wv[s, 1, gsl] = wy0 * wx1
                wv[s, 2, gsl] = wy1 * wx0
                wv[s, 3, gsl] = wy1 * wx1
            for c in range(4):
                pltpu.async_copy(table.at[idx.at[s, c]],
                                 rows.at[pl.ds((s * 4 + c) * SB, SB)],
                                 gsem[s])

    def wait_acc_store(t, s):
        @pl.when(t < cnt)
        def _():
            blk = start + t
            for c in range(4):
                pltpu.make_async_copy(
                    table.at[idx.at[s, c]],
                    rows.at[pl.ds((s * 4 + c) * SB, SB)],
                    gsem[s]).wait()

            @pl.when(t >= 2)
            def _():
                pltpu.make_async_copy(out_v.at[s], out.at[pl.ds(0, SB)],
                                      osem[s]).wait()

            @pl.loop(0, SB // 16)
            def acc_group(g):
                w0 = wv[s, 0, pl.ds(g * 16, 16)]
                w1 = wv[s, 1, pl.ds(g * 16, 16)]
                w2 = wv[s, 2, pl.ds(g * 16, 16)]
                w3 = wv[s, 3, pl.ds(g * 16, 16)]
                for q in range(16):
                    qq = g * 16 + q
                    for h in range(C // 16):
                        hsl = pl.ds(h * 16, 16)
                        out_v[s, qq, hsl] = (
                            rows[(s * 4 + 0) * SB + qq, hsl] * w0[q]
                            + rows[(s * 4 + 1) * SB + qq, hsl] * w1[q]
                            + rows[(s * 4 + 2) * SB + qq, hsl] * w2[q]
                            + rows[(s * 4 + 3) * SB + qq, hsl] * w3[q])

            pltpu.async_copy(out_v.at[s], out.at[pl.ds(blk * SB, SB)],
                             osem[s])

    fire(0, 0)

    @pl.loop(0, ROUNDS, step=2)
    def steady(t):
        fire(t + 1, 1)
        wait_acc_store(t, 0)
        fire(t + 2, 0)
        wait_acc_store(t + 1, 1)

    for s in range(2):
        pltpu.make_async_copy(out_v.at[s], out.at[pl.ds(0, SB)],
                              osem[s]).wait()


@jax.jit
def kernel(x, query_pos):
    table = x.transpose(0, 2, 3, 1).reshape(B * HW, C)
    gx = query_pos[..., 1].reshape(NBLK, SB)
    gy = query_pos[..., 0].reshape(NBLK, SB)
    qxy = jnp.concatenate([gx, gy], axis=1)

    mesh = plsc.VectorSubcoreMesh(core_axis_name="c", subcore_axis_name="s")
    run = functools.partial(
        pl.kernel,
        mesh=mesh,
        out_type=jax.ShapeDtypeStruct((NQ, C), jnp.float32),
        compiler_params=pltpu.CompilerParams(
            use_tc_tiling_on_sc=False, needs_layout_passes=False),
        scratch_types=[
            pltpu.VMEM((ROUNDS, 2 * SB), jnp.float32),
            pltpu.VMEM((2, 4, SB), jnp.int32),
            pltpu.VMEM((2, 4, SB), jnp.float32),
            pltpu.VMEM((2 * 4 * SB, C), jnp.float32),
            pltpu.VMEM((2, SB, C), jnp.float32),
            pltpu.SemaphoreType.DMA,
            pltpu.SemaphoreType.DMA,
            pltpu.SemaphoreType.DMA,
            pltpu.SemaphoreType.DMA,
        ],
    )(_sc_body)
    return run(table, qxy)
